# baseline (speedup 1.0000x reference)
import functools

import jax
import jax.numpy as jnp
from jax import lax
from jax.experimental import pallas as pl
from jax.experimental.pallas import tpu as pltpu
from jax.experimental.pallas import tpu_sc as plsc

_B = 1024
_E = 200
_EMB = 64
_NCH = 3
_NW = 32
_NUNITS = _E * _NCH
_U_LO = _NUNITS // _NW
_U_EXTRA = _NUNITS % _NW
_VOCAB = 129 + 129 + 106
_WORDS = _EMB // 2
_WSTRIDE = _WORDS + 1
_WFLAT = _VOCAB * _WSTRIDE


@functools.partial(
    pl.kernel,
    mesh=plsc.VectorSubcoreMesh(core_axis_name="c", subcore_axis_name="s"),
    out_type=jax.ShapeDtypeStruct((_E, _NCH, 8, 8, 8, 128), jnp.float32),
    compiler_params=pltpu.CompilerParams(
        use_tc_tiling_on_sc=False, needs_layout_passes=False
    ),
    scratch_types=[
        pltpu.VMEM((_WFLAT,), jnp.int32),
        pltpu.VMEM((4, 8, 8, 128), jnp.int32),
        pltpu.VMEM((2, 8, 4, 8, 128), jnp.float32),
        pltpu.SemaphoreType.DMA,
        pltpu.SemaphoreType.DMA,
        pltpu.SemaphoreType.DMA,
    ],
)
def _embed(x4, w1d, out6, wv, idxa, buf, sem0, sem1, semi):
    wid = lax.axis_index("s") * 2 + lax.axis_index("c")

    n_u = _U_LO + (wid < _U_EXTRA).astype(jnp.int32)
    start = _U_LO * wid + jnp.minimum(wid, _U_EXTRA)
    sems = (sem0, sem1)

    f = start // 8
    idx_handles = [
        pltpu.async_copy(x4.at[jnp.minimum(f + k, 74)], idxa.at[k], semi)
        for k in range(4)
    ]
    pltpu.sync_copy(w1d, wv)
    for h in idx_handles:
        h.wait()

    def unit_body(u, _):
        p = start + u
        c = p // _E
        e = p - c * _E
        e8 = e // 8
        e0 = e - e8 * 8
        bl = (c * 25 + e8) - f
        coff = c * 129 * _WSTRIDE

        for h in range(2):
            @pl.when(u >= 1)
            def _():
                pltpu.make_async_copy(
                    buf.at[h],
                    out6.at[e, c, :, pl.ds(h * 4, 4), :, :],
                    sems[h],
                ).wait()

            for b7h in range(4):
                b7 = h * 4 + b7h

                def g_body(g, _, b7=b7, b7h=b7h, h=h):
                    iv = idxa[bl, b7, e0, pl.ds(g * 16, 16)]
                    base = iv * _WSTRIDE + coff
                    for w0 in range(0, _WORDS, 4):
                        vs = [
                            plsc.load_gather(wv, [base + (w0 + t)])
                            for t in range(4)
                        ]
                        for t in range(4):
                            lo, hi = plsc.unpack(
                                plsc.bitcast(vs[t], jnp.bfloat16),
                                format=plsc.PackFormat.INTERLEAVED,
                            )
                            j = 2 * (w0 + t)
                            buf[h, j // 8, b7h, j % 8, pl.ds(g * 16, 16)] = lo
                            j += 1
                            buf[h, j // 8, b7h, j % 8, pl.ds(g * 16, 16)] = hi
                    return 0

                lax.fori_loop(0, 8, g_body, 0)

            pltpu.async_copy(
                buf.at[h], out6.at[e, c, :, pl.ds(h * 4, 4), :, :], sems[h]
            )
        return 0

    lax.fori_loop(0, n_u, unit_body, 0)
    for h in range(2):
        pltpu.make_async_copy(
            buf.at[h], out6.at[0, 0, :, pl.ds(h * 4, 4), :, :], sems[h]
        ).wait()


def kernel(x, W0, W1, W2):
    wf = jnp.concatenate([W0, W1, W2], axis=0)
    w16 = wf.astype(jnp.bfloat16).reshape(_VOCAB, _WORDS, 2)
    wp = jax.lax.bitcast_convert_type(w16, jnp.int32)
    w = jnp.pad(wp, ((0, 0), (0, 1))).reshape(_WFLAT)
    x4 = jnp.transpose(x.reshape(8, 128, 25, 8, 3), (4, 2, 0, 3, 1)).reshape(
        75, 8, 8, 128
    )
    z = _embed(x4, w)
    out = jnp.transpose(z, (3, 5, 0, 1, 2, 4))
    return out.reshape(_B, _E, _NCH, _EMB)


# speedup vs baseline: 1.1019x; 1.1019x over previous
import functools

import jax
import jax.numpy as jnp
from jax import lax
from jax.experimental import pallas as pl
from jax.experimental.pallas import tpu as pltpu
from jax.experimental.pallas import tpu_sc as plsc

_B = 1024
_E = 200
_EMB = 64
_NCH = 3
_NW = 32
_NUNITS = _E * _NCH
_U_LO = _NUNITS // _NW
_U_EXTRA = _NUNITS % _NW
_VOCAB = 129 + 129 + 106
_WSTRIDE = _EMB + 1
_WFLAT = _VOCAB * _WSTRIDE


@functools.partial(
    pl.kernel,
    mesh=plsc.VectorSubcoreMesh(core_axis_name="c", subcore_axis_name="s"),
    out_type=jax.ShapeDtypeStruct((_E, _NCH, 8, 8, 8, 128), jnp.float32),
    compiler_params=pltpu.CompilerParams(
        use_tc_tiling_on_sc=False, needs_layout_passes=False
    ),
    scratch_types=[
        pltpu.VMEM((_WFLAT,), jnp.float32),
        pltpu.VMEM((4, 8, 8, 128), jnp.int32),
        pltpu.VMEM((2, 8, 4, 8, 128), jnp.float32),
        pltpu.SemaphoreType.DMA,
        pltpu.SemaphoreType.DMA,
        pltpu.SemaphoreType.DMA,
    ],
)
def _embed(x4, w1d, out6, wv, idxa, buf, sem0, sem1, semi):
    wid = lax.axis_index("s") * 2 + lax.axis_index("c")

    n_u = _U_LO + (wid < _U_EXTRA).astype(jnp.int32)
    start = _U_LO * wid + jnp.minimum(wid, _U_EXTRA)
    sems = (sem0, sem1)

    f = start // 8
    idx_handles = [
        pltpu.async_copy(x4.at[jnp.minimum(f + k, 74)], idxa.at[k], semi)
        for k in range(4)
    ]
    pltpu.sync_copy(w1d, wv)
    for h in idx_handles:
        h.wait()

    def unit_body(u, _):
        p = start + u
        c = p // _E
        e = p - c * _E
        e8 = e // 8
        e0 = e - e8 * 8
        bl = (c * 25 + e8) - f
        coff = c * 129 * _WSTRIDE

        for h in range(2):
            @pl.when(u >= 1)
            def _():
                pltpu.make_async_copy(
                    buf.at[h],
                    out6.at[e, c, :, pl.ds(h * 4, 4), :, :],
                    sems[h],
                ).wait()

            for b7h in range(4):
                b7 = h * 4 + b7h

                def g_body(g, _, b7=b7, b7h=b7h, h=h):
                    iv = idxa[bl, b7, e0, pl.ds(g * 16, 16)]
                    base = iv * _WSTRIDE + coff
                    for j0 in range(0, _EMB, 16):
                        vs = [
                            plsc.load_gather(wv, [base + (j0 + t)])
                            for t in range(16)
                        ]
                        for t in range(16):
                            j = j0 + t
                            buf[h, j // 8, b7h, j % 8, pl.ds(g * 16, 16)] = vs[t]
                    return 0

                lax.fori_loop(0, 8, g_body, 0)

            pltpu.async_copy(
                buf.at[h], out6.at[e, c, :, pl.ds(h * 4, 4), :, :], sems[h]
            )
        return 0

    lax.fori_loop(0, n_u, unit_body, 0)
    for h in range(2):
        pltpu.make_async_copy(
            buf.at[h], out6.at[0, 0, :, pl.ds(h * 4, 4), :, :], sems[h]
        ).wait()


def kernel(x, W0, W1, W2):
    w = jnp.pad(
        jnp.concatenate([W0, W1, W2], axis=0), ((0, 0), (0, 1))
    ).reshape(_WFLAT)
    x4 = jnp.transpose(x.reshape(8, 128, 25, 8, 3), (4, 2, 0, 3, 1)).reshape(
        75, 8, 8, 128
    )
    z = _embed(x4, w)
    out = jnp.transpose(z, (3, 5, 0, 1, 2, 4))
    return out.reshape(_B, _E, _NCH, _EMB)


# speedup vs baseline: 1.1094x; 1.0068x over previous
import functools

import jax
import jax.numpy as jnp
from jax import lax
from jax.experimental import pallas as pl
from jax.experimental.pallas import tpu as pltpu
from jax.experimental.pallas import tpu_sc as plsc

_B = 1024
_E = 200
_EMB = 64
_NCH = 3
_NW = 32
_NUNITS = _E * _NCH
_U_LO = _NUNITS // _NW
_U_EXTRA = _NUNITS % _NW
_VOCAB = 129 + 129 + 106
_WSTRIDE = _EMB + 1
_WFLAT = _VOCAB * _WSTRIDE


@functools.partial(
    pl.kernel,
    mesh=plsc.VectorSubcoreMesh(core_axis_name="c", subcore_axis_name="s"),
    out_type=jax.ShapeDtypeStruct((_E, _NCH, 8, 8, 8, 128), jnp.float32),
    compiler_params=pltpu.CompilerParams(
        use_tc_tiling_on_sc=False, needs_layout_passes=False
    ),
    scratch_types=[
        pltpu.VMEM((_WFLAT,), jnp.float32),
        pltpu.VMEM((4, 8, 8, 128), jnp.int32),
        pltpu.VMEM((2, 8, 4, 8, 128), jnp.float32),
        pltpu.SemaphoreType.DMA,
        pltpu.SemaphoreType.DMA,
        pltpu.SemaphoreType.DMA,
    ],
)
def _embed(x4, w1d, out6, wv, idxa, buf, sem0, sem1, semi):
    wid = lax.axis_index("s") * 2 + lax.axis_index("c")

    n_u = _U_LO + (wid < _U_EXTRA).astype(jnp.int32)
    start = _U_LO * wid + jnp.minimum(wid, _U_EXTRA)
    sems = (sem0, sem1)

    f = start // 8
    idx_handles = [
        pltpu.async_copy(x4.at[jnp.minimum(f + k, 74)], idxa.at[k], semi)
        for k in range(4)
    ]
    pltpu.sync_copy(w1d, wv)
    for h in idx_handles:
        h.wait()

    def unit_body(u, _):
        p = start + u
        c = p // _E
        e = p - c * _E
        e8 = e // 8
        e0 = e - e8 * 8
        bl = (c * 25 + e8) - f
        coff = c * 129 * _WSTRIDE

        for h in range(2):
            @pl.when(u >= 1)
            def _():
                pltpu.make_async_copy(
                    buf.at[h],
                    out6.at[e, c, :, pl.ds(h * 4, 4), :, :],
                    sems[h],
                ).wait()

            for b7h in range(4):
                b7 = h * 4 + b7h

                def g_body(g, _, b7=b7, b7h=b7h, h=h):
                    iv = idxa[bl, b7, e0, pl.ds(g * 16, 16)]
                    base = iv * _WSTRIDE + coff
                    for j0 in range(0, _EMB, 8):
                        vs = [
                            plsc.load_gather(wv, [base + (j0 + t)])
                            for t in range(8)
                        ]
                        for t in range(8):
                            j = j0 + t
                            buf[h, j // 8, b7h, j % 8, pl.ds(g * 16, 16)] = vs[t]
                    return 0

                lax.fori_loop(0, 8, g_body, 0)

            pltpu.async_copy(
                buf.at[h], out6.at[e, c, :, pl.ds(h * 4, 4), :, :], sems[h]
            )
        return 0

    lax.fori_loop(0, n_u, unit_body, 0)
    for h in range(2):
        pltpu.make_async_copy(
            buf.at[h], out6.at[0, 0, :, pl.ds(h * 4, 4), :, :], sems[h]
        ).wait()


def kernel(x, W0, W1, W2):
    w = jnp.pad(
        jnp.concatenate([W0, W1, W2], axis=0), ((0, 0), (0, 1))
    ).reshape(_WFLAT)
    x4 = jnp.transpose(x.reshape(8, 128, 25, 8, 3), (4, 2, 0, 3, 1)).reshape(
        75, 8, 8, 128
    )
    z = _embed(x4, w)
    out = jnp.transpose(z, (3, 5, 0, 1, 2, 4))
    return out.reshape(_B, _E, _NCH, _EMB)
